# initial kernel scaffold (unmeasured)
import jax
import jax.numpy as jnp
from jax import lax
from jax.experimental import pallas as pl
from jax.experimental.pallas import tpu as pltpu

N_DEV = 4
M = 8192
K = 2048
N = 4096
CH = M // N_DEV


def _gemm_body(x_ref, w_ref, out_ref):
    out_ref[...] = jnp.dot(
        x_ref[...], w_ref[...], preferred_element_type=jnp.float32
    ).astype(jnp.bfloat16)


def _gemm(xb, wb):
    bm = 512
    grid = (M // bm,)
    return pl.pallas_call(
        _gemm_body,
        grid=grid,
        in_specs=[
            pl.BlockSpec((bm, K), lambda i: (i, 0)),
            pl.BlockSpec(memory_space=pltpu.VMEM),
        ],
        out_specs=pl.BlockSpec((bm, N), lambda i: (i, 0)),
        out_shape=jax.ShapeDtypeStruct((M, N), jnp.bfloat16),
    )(xb, wb)


def _rs_body(p_ref, out_ref, comm_ref, local_ref, fstage_ref,
             send_sems, recv_sems, copy_sem):
    my = lax.axis_index("i")
    left = (my + N_DEV - 1) % N_DEV
    right = (my + 1) % N_DEV

    barrier_sem = pltpu.get_barrier_semaphore()
    for nbr in (left, right):
        pl.semaphore_signal(
            barrier_sem, inc=1,
            device_id=(nbr,), device_id_type=pl.DeviceIdType.MESH,
        )
    pl.semaphore_wait(barrier_sem, 2)

    c0 = (my + N_DEV - 1) % N_DEV
    cp = pltpu.make_async_copy(
        p_ref.at[pl.ds(c0 * CH, CH), :], comm_ref.at[0], copy_sem
    )
    cp.start()
    cp.wait()

    for h in range(N_DEV - 1):
        s_slot = h % 2
        r_slot = (h + 1) % 2
        rdma = pltpu.make_async_remote_copy(
            src_ref=comm_ref.at[s_slot],
            dst_ref=comm_ref.at[r_slot],
            send_sem=send_sems.at[s_slot],
            recv_sem=recv_sems.at[r_slot],
            device_id=(right,),
            device_id_type=pl.DeviceIdType.MESH,
        )
        rdma.start()

        c = (my + 2 * N_DEV - 2 - h) % N_DEV
        cp = pltpu.make_async_copy(
            p_ref.at[pl.ds(c * CH, CH), :], local_ref, copy_sem
        )
        cp.start()
        cp.wait()
        rdma.wait()

        if h < N_DEV - 2:
            comm_ref[r_slot] = (
                comm_ref[r_slot][...].astype(jnp.float32)
                + local_ref[...].astype(jnp.float32)
            ).astype(jnp.bfloat16)
        else:
            qn = N // 4
            for q in range(4):
                sl = pl.ds(q * qn, qn)
                fstage_ref[...] = jnp.maximum(
                    comm_ref[r_slot, :, sl].astype(jnp.float32)
                    + local_ref[:, sl].astype(jnp.float32),
                    0.0,
                )
                ocp = pltpu.make_async_copy(
                    fstage_ref, out_ref.at[:, sl], copy_sem
                )
                ocp.start()
                ocp.wait()


def _reduce_scatter(partial):
    return pl.pallas_call(
        _rs_body,
        in_specs=[pl.BlockSpec(memory_space=pl.ANY)],
        out_specs=pl.BlockSpec(memory_space=pl.ANY),
        out_shape=jax.ShapeDtypeStruct((CH, N), jnp.float32),
        scratch_shapes=[
            pltpu.VMEM((2, CH, N), jnp.bfloat16),
            pltpu.VMEM((CH, N), jnp.bfloat16),
            pltpu.VMEM((CH, N // 4), jnp.float32),
            pltpu.SemaphoreType.DMA((2,)),
            pltpu.SemaphoreType.DMA((2,)),
            pltpu.SemaphoreType.DMA,
        ],
        compiler_params=pltpu.CompilerParams(collective_id=0),
    )(partial)


def kernel(x, w_mat):
    xb = x.astype(jnp.bfloat16)
    wb = w_mat.astype(jnp.bfloat16)
    partial = _gemm(xb, wb)
    return _reduce_scatter(partial)


# baseline (device time: 804551 ns/iter reference)
import jax
import jax.numpy as jnp
from jax import lax
from jax.experimental import pallas as pl
from jax.experimental.pallas import tpu as pltpu

N_DEV = 4
M = 8192
K = 2048
N = 4096
CH = M // N_DEV


def _gemm_body(x_ref, w_ref, out_ref):
    out_ref[...] = jnp.dot(
        x_ref[...], w_ref[...], preferred_element_type=jnp.float32
    ).astype(jnp.bfloat16)


def _gemm(xb, wb):
    bm = 512
    grid = (M // bm,)
    return pl.pallas_call(
        _gemm_body,
        grid=grid,
        in_specs=[
            pl.BlockSpec((bm, K), lambda i: (i, 0)),
            pl.BlockSpec(memory_space=pltpu.VMEM),
        ],
        out_specs=pl.BlockSpec((bm, N), lambda i: (i, 0)),
        out_shape=jax.ShapeDtypeStruct((M, N), jnp.bfloat16),
        compiler_params=pltpu.CompilerParams(
            vmem_limit_bytes=100 * 1024 * 1024,
        ),
    )(xb, wb)


def _rs_body(p_ref, out_ref, comm_ref, local_ref, fstage_ref,
             send_sems, recv_sems, copy_sem):
    my = lax.axis_index("i")
    left = (my + N_DEV - 1) % N_DEV
    right = (my + 1) % N_DEV

    barrier_sem = pltpu.get_barrier_semaphore()
    for nbr in (left, right):
        pl.semaphore_signal(
            barrier_sem, inc=1,
            device_id=(nbr,), device_id_type=pl.DeviceIdType.MESH,
        )
    pl.semaphore_wait(barrier_sem, 2)

    c0 = (my + N_DEV - 1) % N_DEV
    cp = pltpu.make_async_copy(
        p_ref.at[pl.ds(c0 * CH, CH), :], comm_ref.at[0], copy_sem
    )
    cp.start()
    cp.wait()

    for h in range(N_DEV - 1):
        s_slot = h % 2
        r_slot = (h + 1) % 2
        rdma = pltpu.make_async_remote_copy(
            src_ref=comm_ref.at[s_slot],
            dst_ref=comm_ref.at[r_slot],
            send_sem=send_sems.at[s_slot],
            recv_sem=recv_sems.at[r_slot],
            device_id=(right,),
            device_id_type=pl.DeviceIdType.MESH,
        )
        rdma.start()

        c = (my + 2 * N_DEV - 2 - h) % N_DEV
        cp = pltpu.make_async_copy(
            p_ref.at[pl.ds(c * CH, CH), :], local_ref, copy_sem
        )
        cp.start()
        cp.wait()
        rdma.wait()

        if h < N_DEV - 2:
            comm_ref[r_slot] = (
                comm_ref[r_slot][...].astype(jnp.float32)
                + local_ref[...].astype(jnp.float32)
            ).astype(jnp.bfloat16)
        else:
            qn = N // 4
            for q in range(4):
                sl = pl.ds(q * qn, qn)
                fstage_ref[...] = jnp.maximum(
                    comm_ref[r_slot, :, sl].astype(jnp.float32)
                    + local_ref[:, sl].astype(jnp.float32),
                    0.0,
                )
                ocp = pltpu.make_async_copy(
                    fstage_ref, out_ref.at[:, sl], copy_sem
                )
                ocp.start()
                ocp.wait()


def _reduce_scatter(partial):
    return pl.pallas_call(
        _rs_body,
        in_specs=[pl.BlockSpec(memory_space=pl.ANY)],
        out_specs=pl.BlockSpec(memory_space=pl.ANY),
        out_shape=jax.ShapeDtypeStruct((CH, N), jnp.float32),
        scratch_shapes=[
            pltpu.VMEM((2, CH, N), jnp.bfloat16),
            pltpu.VMEM((CH, N), jnp.bfloat16),
            pltpu.VMEM((CH, N // 4), jnp.float32),
            pltpu.SemaphoreType.DMA((2,)),
            pltpu.SemaphoreType.DMA((2,)),
            pltpu.SemaphoreType.DMA,
        ],
        compiler_params=pltpu.CompilerParams(
            collective_id=0,
            vmem_limit_bytes=100 * 1024 * 1024,
        ),
    )(partial)


def kernel(x, w_mat):
    xb = x.astype(jnp.bfloat16)
    wb = w_mat.astype(jnp.bfloat16)
    partial = _gemm(xb, wb)
    return _reduce_scatter(partial)


# device time: 502331 ns/iter; 1.6016x vs baseline; 1.6016x over previous
import jax
import jax.numpy as jnp
from jax import lax
from jax.experimental import pallas as pl
from jax.experimental.pallas import tpu as pltpu

N_DEV = 4
M = 8192
K = 2048
N = 4096
CH = M // N_DEV
HW = N // 2


def _gemm_body(x_ref, w_ref, out_ref):
    out_ref[...] = jnp.dot(
        x_ref[...].astype(jnp.bfloat16),
        w_ref[...],
        preferred_element_type=jnp.float32,
    ).astype(jnp.bfloat16)


def _gemm(x, wb):
    bm = 512
    grid = (M // bm,)
    return pl.pallas_call(
        _gemm_body,
        grid=grid,
        in_specs=[
            pl.BlockSpec((bm, K), lambda i: (i, 0)),
            pl.BlockSpec(memory_space=pltpu.VMEM),
        ],
        out_specs=pl.BlockSpec((bm, N), lambda i: (i, 0)),
        out_shape=jax.ShapeDtypeStruct((M, N), jnp.bfloat16),
        compiler_params=pltpu.CompilerParams(
            vmem_limit_bytes=100 * 1024 * 1024,
        ),
    )(x, wb)


def _rs_body(p_ref, out_ref, comm_cw, comm_ccw, local_cw, local_ccw,
             fstage_ref, send_cw, recv_cw, send_ccw, recv_ccw, copy_sems):
    my = lax.axis_index("i")
    left = (my + N_DEV - 1) % N_DEV
    right = (my + 1) % N_DEV

    barrier_sem = pltpu.get_barrier_semaphore()
    for nbr in (left, right):
        pl.semaphore_signal(
            barrier_sem, inc=1,
            device_id=(nbr,), device_id_type=pl.DeviceIdType.MESH,
        )
    pl.semaphore_wait(barrier_sem, 2)

    def stage(chunk, col0, dst, sem):
        cp = pltpu.make_async_copy(
            p_ref.at[pl.ds(chunk * CH, CH), pl.ds(col0, HW)], dst, sem
        )
        cp.start()
        return cp

    cp1 = stage((my + N_DEV - 1) % N_DEV, 0, comm_cw.at[0], copy_sems.at[0])
    cp2 = stage((my + 1) % N_DEV, HW, comm_ccw.at[0], copy_sems.at[1])
    cp1.wait()
    cp2.wait()

    for h in range(N_DEV - 1):
        s = h % 2
        r = (h + 1) % 2
        rdma_cw = pltpu.make_async_remote_copy(
            src_ref=comm_cw.at[s], dst_ref=comm_cw.at[r],
            send_sem=send_cw.at[s], recv_sem=recv_cw.at[r],
            device_id=(right,), device_id_type=pl.DeviceIdType.MESH,
        )
        rdma_ccw = pltpu.make_async_remote_copy(
            src_ref=comm_ccw.at[s], dst_ref=comm_ccw.at[r],
            send_sem=send_ccw.at[s], recv_sem=recv_ccw.at[r],
            device_id=(left,), device_id_type=pl.DeviceIdType.MESH,
        )
        rdma_cw.start()
        rdma_ccw.start()

        c_cw = (my + 2 * N_DEV - 2 - h) % N_DEV
        c_ccw = (my + 2 + h) % N_DEV
        cp1 = stage(c_cw, 0, local_cw, copy_sems.at[0])
        cp2 = stage(c_ccw, HW, local_ccw, copy_sems.at[1])
        cp1.wait()
        cp2.wait()

        if h < N_DEV - 2:
            rdma_cw.wait()
            comm_cw[r] = (
                comm_cw[r][...].astype(jnp.float32)
                + local_cw[...].astype(jnp.float32)
            ).astype(jnp.bfloat16)
            rdma_ccw.wait()
            comm_ccw[r] = (
                comm_ccw[r][...].astype(jnp.float32)
                + local_ccw[...].astype(jnp.float32)
            ).astype(jnp.bfloat16)
        else:
            qn = HW // 2
            rdma_cw.wait()
            for q in range(2):
                sl = pl.ds(q * qn, qn)
                fstage_ref[...] = jnp.maximum(
                    comm_cw[r, :, sl].astype(jnp.float32)
                    + local_cw[:, sl].astype(jnp.float32),
                    0.0,
                )
                ocp = pltpu.make_async_copy(
                    fstage_ref, out_ref.at[:, pl.ds(q * qn, qn)],
                    copy_sems.at[0],
                )
                ocp.start()
                ocp.wait()
            rdma_ccw.wait()
            for q in range(2):
                sl = pl.ds(q * qn, qn)
                fstage_ref[...] = jnp.maximum(
                    comm_ccw[r, :, sl].astype(jnp.float32)
                    + local_ccw[:, sl].astype(jnp.float32),
                    0.0,
                )
                ocp = pltpu.make_async_copy(
                    fstage_ref, out_ref.at[:, pl.ds(HW + q * qn, qn)],
                    copy_sems.at[0],
                )
                ocp.start()
                ocp.wait()


def _reduce_scatter(partial):
    return pl.pallas_call(
        _rs_body,
        in_specs=[pl.BlockSpec(memory_space=pl.ANY)],
        out_specs=pl.BlockSpec(memory_space=pl.ANY),
        out_shape=jax.ShapeDtypeStruct((CH, N), jnp.float32),
        scratch_shapes=[
            pltpu.VMEM((2, CH, HW), jnp.bfloat16),
            pltpu.VMEM((2, CH, HW), jnp.bfloat16),
            pltpu.VMEM((CH, HW), jnp.bfloat16),
            pltpu.VMEM((CH, HW), jnp.bfloat16),
            pltpu.VMEM((CH, HW // 2), jnp.float32),
            pltpu.SemaphoreType.DMA((2,)),
            pltpu.SemaphoreType.DMA((2,)),
            pltpu.SemaphoreType.DMA((2,)),
            pltpu.SemaphoreType.DMA((2,)),
            pltpu.SemaphoreType.DMA((2,)),
        ],
        compiler_params=pltpu.CompilerParams(
            collective_id=0,
            vmem_limit_bytes=100 * 1024 * 1024,
        ),
    )(partial)


def kernel(x, w_mat):
    wb = w_mat.astype(jnp.bfloat16)
    partial = _gemm(x, wb)
    return _reduce_scatter(partial)


# device time: 492623 ns/iter; 1.6332x vs baseline; 1.0197x over previous
import jax
import jax.numpy as jnp
from jax import lax
from jax.experimental import pallas as pl
from jax.experimental.pallas import tpu as pltpu

N_DEV = 4
M = 8192
K = 2048
N = 4096
CH = M // N_DEV
HW = N // 2


def _gemm_body(x_ref, w_ref, out_ref):
    out_ref[...] = jnp.dot(
        x_ref[...].astype(jnp.bfloat16),
        w_ref[...],
        preferred_element_type=jnp.float32,
    ).astype(jnp.bfloat16)


def _gemm(x, wb):
    bm = 512
    grid = (M // bm,)
    return pl.pallas_call(
        _gemm_body,
        grid=grid,
        in_specs=[
            pl.BlockSpec((bm, K), lambda i: (i, 0)),
            pl.BlockSpec(memory_space=pltpu.VMEM),
        ],
        out_specs=pl.BlockSpec((bm, N), lambda i: (i, 0)),
        out_shape=jax.ShapeDtypeStruct((M, N), jnp.bfloat16),
        compiler_params=pltpu.CompilerParams(
            vmem_limit_bytes=100 * 1024 * 1024,
        ),
    )(x, wb)


def _rs_body(p_ref, out_ref, comm_cw, comm_ccw, local_cw, local_ccw,
             fstage_ref, send_cw, recv_cw, send_ccw, recv_ccw, copy_sems):
    my = lax.axis_index("i")
    left = (my + N_DEV - 1) % N_DEV
    right = (my + 1) % N_DEV

    barrier_sem = pltpu.get_barrier_semaphore()
    for nbr in (left, right):
        pl.semaphore_signal(
            barrier_sem, inc=1,
            device_id=(nbr,), device_id_type=pl.DeviceIdType.MESH,
        )
    pl.semaphore_wait(barrier_sem, 2)

    def stage(chunk, col0, dst, sem):
        cp = pltpu.make_async_copy(
            p_ref.at[pl.ds(chunk * CH, CH), pl.ds(col0, HW)], dst, sem
        )
        cp.start()
        return cp

    cp1 = stage((my + N_DEV - 1) % N_DEV, 0, comm_cw.at[0], copy_sems.at[0])
    cp2 = stage((my + 1) % N_DEV, HW, comm_ccw.at[0], copy_sems.at[1])
    cp1.wait()
    cp2.wait()

    def mk_rdma_cw(h):
        return pltpu.make_async_remote_copy(
            src_ref=comm_cw.at[h % 2], dst_ref=comm_cw.at[(h + 1) % 2],
            send_sem=send_cw.at[h % 2], recv_sem=recv_cw.at[(h + 1) % 2],
            device_id=(right,), device_id_type=pl.DeviceIdType.MESH,
        )

    def mk_rdma_ccw(h):
        return pltpu.make_async_remote_copy(
            src_ref=comm_ccw.at[h % 2], dst_ref=comm_ccw.at[(h + 1) % 2],
            send_sem=send_ccw.at[h % 2], recv_sem=recv_ccw.at[(h + 1) % 2],
            device_id=(left,), device_id_type=pl.DeviceIdType.MESH,
        )

    rdma_cw = mk_rdma_cw(0)
    rdma_ccw = mk_rdma_ccw(0)
    rdma_cw.start()
    rdma_ccw.start()
    cp1 = stage((my + 2 * N_DEV - 2) % N_DEV, 0, local_cw, copy_sems.at[0])
    cp2 = stage((my + 2) % N_DEV, HW, local_ccw, copy_sems.at[1])

    for h in range(N_DEV - 2):
        r = (h + 1) % 2
        cp1.wait()
        rdma_cw.wait()
        comm_cw[r] = (
            comm_cw[r][...].astype(jnp.float32)
            + local_cw[...].astype(jnp.float32)
        ).astype(jnp.bfloat16)
        next_cw = mk_rdma_cw(h + 1)
        next_cw.start()

        cp2.wait()
        rdma_ccw.wait()
        comm_ccw[r] = (
            comm_ccw[r][...].astype(jnp.float32)
            + local_ccw[...].astype(jnp.float32)
        ).astype(jnp.bfloat16)
        next_ccw = mk_rdma_ccw(h + 1)
        next_ccw.start()

        cp1 = stage((my + 2 * N_DEV - 3 - h) % N_DEV, 0, local_cw,
                    copy_sems.at[0])
        cp2 = stage((my + 3 + h) % N_DEV, HW, local_ccw, copy_sems.at[1])
        rdma_cw, rdma_ccw = next_cw, next_ccw

    r = (N_DEV - 1) % 2
    qn = 512
    odesc = [None, None]

    def finalize(comm, local, col0):
        for q in range(HW // qn):
            b = q % 2
            sl = pl.ds(q * qn, qn)
            if odesc[b] is not None:
                odesc[b].wait()
            fstage_ref[b] = jnp.maximum(
                comm[r, :, sl].astype(jnp.float32)
                + local[:, sl].astype(jnp.float32),
                0.0,
            )
            od = pltpu.make_async_copy(
                fstage_ref.at[b], out_ref.at[:, pl.ds(col0 + q * qn, qn)],
                copy_sems.at[b],
            )
            od.start()
            odesc[b] = od

    cp1.wait()
    cp2.wait()
    rdma_cw.wait()
    finalize(comm_cw, local_cw, 0)
    rdma_ccw.wait()
    finalize(comm_ccw, local_ccw, HW)
    for od in odesc:
        od.wait()


def _reduce_scatter(partial):
    return pl.pallas_call(
        _rs_body,
        in_specs=[pl.BlockSpec(memory_space=pl.ANY)],
        out_specs=pl.BlockSpec(memory_space=pl.ANY),
        out_shape=jax.ShapeDtypeStruct((CH, N), jnp.float32),
        scratch_shapes=[
            pltpu.VMEM((2, CH, HW), jnp.bfloat16),
            pltpu.VMEM((2, CH, HW), jnp.bfloat16),
            pltpu.VMEM((CH, HW), jnp.bfloat16),
            pltpu.VMEM((CH, HW), jnp.bfloat16),
            pltpu.VMEM((2, CH, 512), jnp.float32),
            pltpu.SemaphoreType.DMA((2,)),
            pltpu.SemaphoreType.DMA((2,)),
            pltpu.SemaphoreType.DMA((2,)),
            pltpu.SemaphoreType.DMA((2,)),
            pltpu.SemaphoreType.DMA((2,)),
        ],
        compiler_params=pltpu.CompilerParams(
            collective_id=0,
            vmem_limit_bytes=100 * 1024 * 1024,
        ),
    )(partial)


def kernel(x, w_mat):
    wb = w_mat.astype(jnp.bfloat16)
    partial = _gemm(x, wb)
    return _reduce_scatter(partial)
